# baseline (speedup 1.0000x reference)
import functools

import jax
import jax.numpy as jnp
from jax import lax
from jax.experimental import pallas as pl
from jax.experimental.pallas import tpu as pltpu
from jax.experimental.pallas import tpu_sc as plsc

N = 100
D = 131072
S = 128
LOG2_S = 7
D2 = D // S
B2 = N * S
NW = 32
PER_W = B2 // NW
K = 16
STEPS = PER_W // K
L = 16

_mesh = plsc.VectorSubcoreMesh(core_axis_name="c", subcore_axis_name="s")


@functools.partial(
    pl.kernel,
    out_type=jax.ShapeDtypeStruct((B2, D2), jnp.float32),
    mesh=_mesh,
    compiler_params=pltpu.CompilerParams(needs_layout_passes=False),
    scratch_types=[
        pltpu.VMEM((128,), jnp.int32),
        pltpu.VMEM((PER_W,), jnp.int32),
        pltpu.VMEM((K, D2), jnp.float32),
        pltpu.SemaphoreType.DMA,
    ],
)
def _sc_gather(x2, ords, out, ordv, idxv, buf, sem):
    cid = lax.axis_index("c")
    sid = lax.axis_index("s")
    wid = sid * 2 + cid
    base = wid * PER_W

    pltpu.sync_copy(ords, ordv.at[pl.ds(0, N)])

    for j0 in range(0, PER_W, L):
        r = base + j0 + lax.iota(jnp.int32, L)
        i = lax.shift_right_logical(r, LOG2_S)
        c = lax.bitwise_and(r, S - 1)
        ov = plsc.load_gather(ordv, [i])
        idxv[pl.ds(j0, L)] = ov * S + c

    for s in range(STEPS):
        pltpu.async_copy(x2.at[idxv.at[pl.ds(s * K, K)]], buf, sem).wait()
        pltpu.sync_copy(buf, out.at[pl.ds(base + s * K, K)])


def kernel(x, ordinals):
    x2 = x.reshape(B2, D2)
    out2 = _sc_gather(x2, ordinals)
    return out2.reshape(N, D)


# speedup vs baseline: 1.1368x; 1.1368x over previous
import functools

import jax
import jax.numpy as jnp
from jax import lax
from jax.experimental import pallas as pl
from jax.experimental.pallas import tpu as pltpu
from jax.experimental.pallas import tpu_sc as plsc

N = 100
D = 131072
S = 128
LOG2_S = 7
D2 = D // S
B2 = N * S
NW = 32
PER_W = B2 // NW
K = 16
STEPS = PER_W // K
NBUF = 4
L = 16

_mesh = plsc.VectorSubcoreMesh(core_axis_name="c", subcore_axis_name="s")


@functools.partial(
    pl.kernel,
    out_type=jax.ShapeDtypeStruct((B2, D2), jnp.float32),
    mesh=_mesh,
    compiler_params=pltpu.CompilerParams(needs_layout_passes=False),
    scratch_types=[
        pltpu.VMEM((128,), jnp.int32),
        pltpu.VMEM((PER_W,), jnp.int32),
        [pltpu.VMEM((K, D2), jnp.float32) for _ in range(NBUF)],
        [pltpu.SemaphoreType.DMA for _ in range(NBUF)],
        [pltpu.SemaphoreType.DMA for _ in range(NBUF)],
    ],
)
def _sc_gather(x2, ords, out, ordv, idxv, bufs, gsems, wsems):
    cid = lax.axis_index("c")
    sid = lax.axis_index("s")
    wid = sid * 2 + cid
    base = wid * PER_W

    pltpu.sync_copy(ords, ordv.at[pl.ds(0, N)])

    for j0 in range(0, PER_W, L):
        r = base + j0 + lax.iota(jnp.int32, L)
        i = lax.shift_right_logical(r, LOG2_S)
        c = lax.bitwise_and(r, S - 1)
        ov = plsc.load_gather(ordv, [i])
        idxv[pl.ds(j0, L)] = ov * S + c

    def start_gather(s):
        b = s % NBUF
        pltpu.async_copy(x2.at[idxv.at[pl.ds(s * K, K)]], bufs[b], gsems[b])

    scats = [None] * NBUF
    for s in range(min(NBUF - 1, STEPS)):
        start_gather(s)
    for s in range(STEPS):
        b = s % NBUF
        pltpu.make_async_copy(x2.at[idxv.at[pl.ds(s * K, K)]],
                              bufs[b], gsems[b]).wait()
        scats[b] = pltpu.async_copy(bufs[b], out.at[pl.ds(base + s * K, K)],
                                    wsems[b])
        n = s + NBUF - 1
        if n < STEPS:
            nb = n % NBUF
            if scats[nb] is not None:
                scats[nb].wait()
                scats[nb] = None
            start_gather(n)
    for b in range(NBUF):
        if scats[b] is not None:
            scats[b].wait()


def kernel(x, ordinals):
    x2 = x.reshape(B2, D2)
    out2 = _sc_gather(x2, ordinals)
    return out2.reshape(N, D)
